# baseline (speedup 1.0000x reference)
import math

import jax
import jax.numpy as jnp
from jax.experimental import pallas as pl
from jax.experimental.pallas import tpu as pltpu

V = 32000
SMOOTH_A = 0.1 / (V - 1)
CONF_C = 0.9
K_CONST = (V - 1) * SMOOTH_A * math.log(SMOOTH_A) + CONF_C * math.log(CONF_C)

NS = 4
R_BLK = 256
V_BLK = 6400
N_ROWS = 4096
QTR = N_ROWS // NS
NR = QTR // R_BLK
NV = V // V_BLK


def _loss_body(tgt_ref, p0, p1, p2, p3, out_ref, acc_ref, cnt_ref):
    i = pl.program_id(0)
    j = pl.program_id(1)

    @pl.when((i == 0) & (j == 0))
    def _init():
        acc_ref[0] = 0.0
        cnt_ref[0] = 0.0

    tgt = tgt_ref[0, 0, :]

    @pl.when(j == 0)
    def _count():
        cnt_ref[0] += jnp.sum((tgt > 0).astype(jnp.float32))

    col0 = jax.lax.broadcasted_iota(jnp.int32, (R_BLK, V_BLK), 1)
    s = 0.0
    for k, ref in enumerate((p0, p1, p2, p3)):
        tg = tgt[k * R_BLK:(k + 1) * R_BLK]
        maskf = (tg > 0).astype(jnp.float32)
        tloc = tg - j * V_BLK
        w = jnp.where(col0 == tloc[:, None], CONF_C, SMOOTH_A)
        row_part = jnp.sum(ref[...] * w, axis=1)
        s = s + jnp.sum(row_part * maskf)
    acc_ref[0] += s

    @pl.when((i == NR - 1) & (j == NV - 1))
    def _fin():
        out_ref[0] = K_CONST - acc_ref[0] / cnt_ref[0]


def kernel(prediction, target):
    pred = prediction.reshape(N_ROWS, V)
    tgt = target.reshape(N_ROWS).astype(jnp.int32)
    tgt2 = jnp.concatenate(
        [tgt[k * QTR:(k + 1) * QTR].reshape(NR, 1, R_BLK) for k in range(NS)],
        axis=2)

    def mk_spec(k):
        return pl.BlockSpec((R_BLK, V_BLK), lambda i, j, k=k: (i + k * NR, j))

    out = pl.pallas_call(
        _loss_body,
        grid=(NR, NV),
        compiler_params=pltpu.CompilerParams(
            vmem_limit_bytes=100 * 1024 * 1024),
        in_specs=[pl.BlockSpec((1, 1, NS * R_BLK), lambda i, j: (i, 0, 0))]
        + [mk_spec(k) for k in range(NS)],
        out_specs=pl.BlockSpec(memory_space=pltpu.SMEM),
        out_shape=jax.ShapeDtypeStruct((1,), jnp.float32),
        scratch_shapes=[
            pltpu.SMEM((1,), jnp.float32),
            pltpu.SMEM((1,), jnp.float32),
        ],
    )(tgt2, pred, pred, pred, pred)
    return out[0]
